# baseline (speedup 1.0000x reference)
import jax
import jax.numpy as jnp
from jax import lax
from jax.experimental import pallas as pl
from jax.experimental.pallas import tpu as pltpu

_N = 20000
_PAD_N = 20480
_ROWS = 160
_LANES = 128
_CONF = 0.25
_IOU_T = 0.45
_MAX_DET = 300
_BIG = 2**30


def _fold(av, ai, bv, bi):
    t = (bv > av) | ((bv == av) & (bi < ai))
    return jnp.where(t, bv, av), jnp.where(t, bi, ai)


def _argmax_pair(s, lin):
    vs = [s[8 * k : 8 * (k + 1)] for k in range(20)]
    ids = [lin[8 * k : 8 * (k + 1)] for k in range(20)]
    n = 20
    while n > 1:
        half = (n + 1) // 2
        for k in range(n // 2):
            vs[k], ids[k] = _fold(vs[k], ids[k], vs[k + half], ids[k + half])
        n = half
    v, i = vs[0], ids[0]
    for sh in (4, 2, 1):
        v2 = pltpu.roll(v, 8 - sh, 0)
        i2 = pltpu.roll(i, 8 - sh, 0)
        v, i = _fold(v, i, v2, i2)
    for sh in (64, 32, 16, 8, 4, 2, 1):
        v2 = pltpu.roll(v, 128 - sh, 1)
        i2 = pltpu.roll(i, 128 - sh, 1)
        v, i = _fold(v, i, v2, i2)
    return v[0, 0], i[0, 0]


def _nms_kernel(planes_ref, scores_ref, out_ref):
    cx = planes_ref[0]
    cy = planes_ref[1]
    w = planes_ref[2]
    h = planes_ref[3]
    x1 = cx - w / 2
    y1 = cy - h / 2
    x2 = cx + w / 2
    y2 = cy + h / 2
    area = jnp.clip(x2 - x1, 0.0) * jnp.clip(y2 - y1, 0.0)

    raw_s = scores_ref[...]
    s0 = jnp.where(raw_s > _CONF, raw_s, 0.0)

    lin = (
        lax.broadcasted_iota(jnp.int32, (_ROWS, _LANES), 0) * _LANES
        + lax.broadcasted_iota(jnp.int32, (_ROWS, _LANES), 1)
    )
    lane_iota = lax.broadcasted_iota(jnp.int32, (1, _LANES), 1)
    lane8 = lax.broadcasted_iota(jnp.int32, (1, 8), 1)

    m0, idx0 = _argmax_pair(s0, lin)

    def body(i, carry):
        s, m, idx = carry
        valid = m > 0.0
        r = idx // _LANES
        c = idx % _LANES
        sel = lane_iota == c
        cxr = planes_ref[0, pl.ds(r, 1), :]
        cyr = planes_ref[1, pl.ds(r, 1), :]
        wr = planes_ref[2, pl.ds(r, 1), :]
        hr = planes_ref[3, pl.ds(r, 1), :]
        bx1 = jnp.sum(jnp.where(sel, cxr - wr / 2, 0.0))
        by1 = jnp.sum(jnp.where(sel, cyr - hr / 2, 0.0))
        bx2 = jnp.sum(jnp.where(sel, cxr + wr / 2, 0.0))
        by2 = jnp.sum(jnp.where(sel, cyr + hr / 2, 0.0))
        ix1 = jnp.maximum(bx1, x1)
        iy1 = jnp.maximum(by1, y1)
        ix2 = jnp.minimum(bx2, x2)
        iy2 = jnp.minimum(by2, y2)
        inter = jnp.clip(ix2 - ix1, 0.0) * jnp.clip(iy2 - iy1, 0.0)
        area_a = jnp.clip(bx2 - bx1, 0.0) * jnp.clip(by2 - by1, 0.0)
        iou = inter / (area_a + area - inter + 1e-9)
        s = jnp.where((iou > _IOU_T) | (lin == idx), 0.0, s)
        m_next, idx_next = _argmax_pair(s, lin)
        vf = jnp.where(valid, 1.0, 0.0)
        row = (
            jnp.where(lane8 == 0, bx1, 0.0)
            + jnp.where(lane8 == 1, by1, 0.0)
            + jnp.where(lane8 == 2, bx2, 0.0)
            + jnp.where(lane8 == 3, by2, 0.0)
            + jnp.where(lane8 == 4, m, 0.0)
        ) * vf
        out_ref[pl.ds(i, 1), :] = row
        return (s, m_next, idx_next)

    lax.fori_loop(0, _MAX_DET, body, (s0, m0, idx0))


def kernel(boxes, scores):
    planes = jnp.pad(boxes, ((0, _PAD_N - _N), (0, 0))).T.reshape(
        4, _ROWS, _LANES
    )
    s = jnp.pad(scores, (0, _PAD_N - _N)).reshape(_ROWS, _LANES)
    out = pl.pallas_call(
        _nms_kernel,
        out_shape=jax.ShapeDtypeStruct((_MAX_DET, 8), jnp.float32),
    )(planes, s)
    return out[:, :5]


# speedup vs baseline: 1.2029x; 1.2029x over previous
import jax
import jax.numpy as jnp
from jax import lax
from jax.experimental import pallas as pl
from jax.experimental.pallas import tpu as pltpu

_N = 20000
_PAD_N = 20480
_ROWS = 160
_LANES = 128
_CONF = 0.25
_IOU_T = 0.45
_MAX_DET = 300
_BIG = 2**30


def _nms_kernel(planes_ref, packed_ref, scores_ref, out_ref, s_ref):
    cx = planes_ref[0]
    cy = planes_ref[1]
    w = planes_ref[2]
    h = planes_ref[3]
    x1 = cx - w / 2
    y1 = cy - h / 2
    x2 = cx + w / 2
    y2 = cy + h / 2
    area = jnp.clip(x2 - x1, 0.0) * jnp.clip(y2 - y1, 0.0)

    raw_s = scores_ref[...]
    s0 = jnp.where(raw_s > _CONF, raw_s, 0.0)
    s_ref[...] = s0

    lin = (
        lax.broadcasted_iota(jnp.int32, (_ROWS, _LANES), 0) * _LANES
        + lax.broadcasted_iota(jnp.int32, (_ROWS, _LANES), 1)
    )
    lane8 = lax.broadcasted_iota(jnp.int32, (1, 8), 1)

    idx0 = jnp.argmax(s0).astype(jnp.int32)

    def body(i, idx):
        g = packed_ref[pl.ds(idx // 2, 1), :]
        hi = idx % 2 == 1
        srow = s_ref[pl.ds(idx // _LANES, 1), :]
        m = jnp.max(
            jnp.where(
                lax.broadcasted_iota(jnp.int32, (1, _LANES), 1)
                == idx % _LANES,
                srow,
                0.0,
            )
        )
        bcx = jnp.where(hi, g[0, 4], g[0, 0])
        bcy = jnp.where(hi, g[0, 5], g[0, 1])
        bw = jnp.where(hi, g[0, 6], g[0, 2])
        bh = jnp.where(hi, g[0, 7], g[0, 3])
        bx1 = bcx - bw / 2
        by1 = bcy - bh / 2
        bx2 = bcx + bw / 2
        by2 = bcy + bh / 2
        s = s_ref[...]
        ix1 = jnp.maximum(bx1, x1)
        iy1 = jnp.maximum(by1, y1)
        ix2 = jnp.minimum(bx2, x2)
        iy2 = jnp.minimum(by2, y2)
        inter = jnp.clip(ix2 - ix1, 0.0) * jnp.clip(iy2 - iy1, 0.0)
        area_a = jnp.clip(bx2 - bx1, 0.0) * jnp.clip(by2 - by1, 0.0)
        iou = inter / (area_a + area - inter + 1e-9)
        s = jnp.where((iou > _IOU_T) | (lin == idx), 0.0, s)
        s_ref[...] = s
        idx_next = jnp.argmax(s).astype(jnp.int32)
        valid = m > 0.0
        vf = jnp.where(valid, 1.0, 0.0)
        row = (
            jnp.where(lane8 == 0, bx1, 0.0)
            + jnp.where(lane8 == 1, by1, 0.0)
            + jnp.where(lane8 == 2, bx2, 0.0)
            + jnp.where(lane8 == 3, by2, 0.0)
            + jnp.where(lane8 == 4, m, 0.0)
        ) * vf
        out_ref[pl.ds(i, 1), :] = row
        return idx_next

    lax.fori_loop(0, _MAX_DET, body, idx0)


def kernel(boxes, scores):
    bp = jnp.pad(boxes, ((0, _PAD_N - _N), (0, 0)))
    planes = bp.T.reshape(4, _ROWS, _LANES)
    packed = bp.reshape(_PAD_N // 2, 8)
    s = jnp.pad(scores, (0, _PAD_N - _N)).reshape(_ROWS, _LANES)
    out = pl.pallas_call(
        _nms_kernel,
        out_shape=jax.ShapeDtypeStruct((_MAX_DET, 8), jnp.float32),
        scratch_shapes=[pltpu.VMEM((_ROWS, _LANES), jnp.float32)],
    )(planes, packed, s)
    return out[:, :5]
